# baseline (speedup 1.0000x reference)
import functools

import jax
import jax.numpy as jnp
from jax import lax
from jax.experimental import pallas as pl
from jax.experimental.pallas import tpu as pltpu
from jax.experimental.pallas import tpu_sc as plsc

_NC = 2
_NS = 16
_B = 16
_NPAD = 10240
_BLK = 1024

_F32 = jnp.float32


def _tdot(a, b):
  return lax.dot_general(a, b, (((0,), (1,)), ((), ())),
                         preferred_element_type=_F32)


def _rowdot(v, c):
  return lax.dot_general(v.reshape(1, -1), c, (((1,), (1,)), ((), ())),
                         preferred_element_type=_F32)


def _compose_body(wcomb, wr1, wr2, wr3, wr4, wr5, wo1, wo2, wo3, wo4, wo5,
                  wfeat, wproj, bfeat, bcomb, bproj, br1, br2, br3, br4, br5,
                  w_all_ref, bias_ref):
  wcomb = wcomb[...]
  cs = [wcomb[:, 128 * j:128 * (j + 1)] for j in range(7)]
  wrels = [wr1[...], wr2[...], wr3[...], wr4[...], wr5[...]]
  wroots = [wo1[...], wo2[...], wo3[...], wo4[...], wo5[...]]
  brels = [br1[...], br2[...], br3[...], br4[...], br5[...]]
  wrels.append(wrels[4])
  wroots.append(wroots[4])
  brels.append(brels[4])

  blocks = [_tdot(wrels[k], cs[k + 1]) for k in range(6)]
  gt = _tdot(wfeat[...], cs[0])
  for k in range(6):
    gt = gt + _tdot(wroots[k], cs[k + 1])
  wp1 = wproj[...][:, :128]
  ft = _tdot(wfeat[...], wp1)
  w_all_ref[...] = jnp.concatenate(blocks + [gt, ft], axis=1)

  cp = bcomb[...].reshape(1, -1) + _rowdot(bfeat[...], cs[0])
  for k in range(6):
    cp = cp + _rowdot(brels[k], cs[k + 1])
  dd = bproj[...].reshape(1, -1) + _rowdot(bfeat[...], wp1)
  bias_ref[...] = jnp.concatenate([cp, dd], axis=0)


def _transform_body(x_ref, w_ref, b_ref, u6_ref, p0_ref, xf_ref):
  out = jnp.dot(x_ref[...], w_ref[...], preferred_element_type=_F32)
  u6_ref[...] = out[:, :768]
  p0_ref[...] = out[:, 768:896] + b_ref[0][None, :]
  xf_ref[...] = out[:, 896:] + b_ref[1][None, :]


def _finish_body(parts_ref, cnts_ref, p0_ref, xf_ref, wp_ref, out_ref):
  s = parts_ref[0] + parts_ref[1]
  c4 = cnts_ref[0] + cnts_ref[1]
  nr = _BLK // 128
  sel = (lax.broadcasted_iota(jnp.int32, (_BLK, nr), 0) // 128 ==
         lax.broadcasted_iota(jnp.int32, (_BLK, nr), 1)).astype(_F32)
  rep = jnp.dot(sel, c4, preferred_element_type=_F32)
  pmod = lax.broadcasted_iota(jnp.int32, (_BLK, 128), 0) % 128
  lane = lax.broadcasted_iota(jnp.int32, (_BLK, 128), 1)
  cnt_col = jnp.sum(jnp.where(pmod == lane, rep, 0.0), axis=1, keepdims=True)
  cnt = jnp.maximum(cnt_col, 1.0)
  proj = jnp.maximum(p0_ref[...] + s / cnt, 0.0)
  o = xf_ref[...] + lax.dot_general(
      proj, wp_ref[...][:, 128:], (((1,), (1,)), ((), ())),
      preferred_element_type=_F32)
  nrm = jnp.sqrt(jnp.sum(o * o, axis=1, keepdims=True))
  out_ref[...] = o / jnp.maximum(nrm, 1e-12)


def _sc_body(n_chunks, u6, edata, zer, out, outc,
             acc, cacc, fbuf0, fbuf1, fbuf2, srcv0, srcv1, srcv2,
             dstv0, dstv1, ddv0, ddv1, rows0, rows1, rows2,
             msg0, msg1, msg20, msg21,
             esem0, esem1, esem2, gsem0, gsem1, gsem2, ssem0, ssem1):
  c = lax.axis_index("c")
  s = lax.axis_index("s")
  w = c * _NS + s
  nw = _NC * _NS
  rpt = _NPAD // _NS
  ncr = _NPAD // 128
  my = n_chunks // nw
  rec = _B * 18

  fbufs = [fbuf0, fbuf1, fbuf2]
  srcvs = [srcv0, srcv1, srcv2]
  rowss = [rows0, rows1, rows2]
  esems = [esem0, esem1, esem2]
  gsems = [gsem0, gsem1, gsem2]
  dstvs = [dstv0, dstv1]
  ddvs = [ddv0, ddv1]
  msgs = [msg0, msg1]
  msg2s = [msg20, msg21]
  ssems = [ssem0, ssem1]

  pltpu.sync_copy(zer.at[pl.ds(s * rpt, rpt)], acc.at[pl.ds(s * rpt, rpt)])

  @pl.when(s < ncr // 8)
  def _zero_counts():
    pltpu.sync_copy(zer.at[pl.ds(s * 8, 8)], cacc.at[pl.ds(s * 8, 8)])

  plsc.subcore_barrier()

  iota16 = lax.convert_element_type(
      lax.broadcasted_iota(jnp.int32, (16,), 0), _F32)

  def issue_idx(t, b):
    lo = (w + t * nw) * rec
    pltpu.async_copy(edata.at[pl.ds(lo, rec)], fbufs[b], esems[b])

  def wait_idx(b):
    pltpu.make_async_copy(edata.at[pl.ds(0, rec)], fbufs[b],
                          esems[b]).wait()

  def issue_gather(b):
    srcvs[b][pl.ds(0, 16)] = lax.convert_element_type(
        fbufs[b][pl.ds(0, 16)], jnp.int32)
    pltpu.async_copy(u6.at[srcvs[b]], rowss[b], gsems[b])

  def wait_gather(b):
    pltpu.make_async_copy(u6.at[srcvs[b]], rowss[b], gsems[b]).wait()

  def issue_scatter(b):
    pltpu.async_copy(msgs[b], acc.at[dstvs[b]], ssems[b], add=True)
    pltpu.async_copy(msg2s[b], cacc.at[ddvs[b]], ssems[b], add=True)

  def wait_scatter(b):
    pltpu.make_async_copy(msgs[b], acc.at[dstvs[b]], ssems[b]).wait()
    pltpu.make_async_copy(msg2s[b], cacc.at[ddvs[b]], ssems[b]).wait()

  issue_idx(0, 0)
  issue_idx(1, 1)
  issue_idx(2, 2)
  wait_idx(0)
  issue_gather(0)
  wait_idx(1)
  issue_gather(1)

  def chunk_step(t, r):
    p3 = r % 3
    p2 = r % 2
    q3 = (r + 2) % 3

    @pl.when(t + 2 < my)
    def _prefetch_gather():
      wait_idx(q3)
      issue_gather(q3)

    wait_gather(p3)

    @pl.when(t >= 2)
    def _drain_scatter():
      wait_scatter(p2)

    dpart = lax.convert_element_type(fbufs[p3][pl.ds(16, 16)], jnp.int32)
    dstvs[p2][pl.ds(0, 16)] = dpart
    ddvs[p2][pl.ds(0, 16)] = lax.shift_right_logical(dpart, 7)

    def _one(i):
      wrow = fbufs[p3][pl.ds(32 + i * 16, 16)]
      dm = wrow[6]
      for j in range(8):
        v = wrow[0] * rowss[p3][i, pl.ds(j * 16, 16)]
        for k in range(1, 6):
          v = v + wrow[k] * rowss[p3][i, pl.ds(k * 128 + j * 16, 16)]
        msgs[p2][i, pl.ds(j * 16, 16)] = v
        msg2s[p2][i, pl.ds(j * 16, 16)] = jnp.where(
            iota16 + (16.0 * j) == dm, 1.0, 0.0)

    def _edge2(i, inner):
      _one(2 * i)
      _one(2 * i + 1)
      return inner

    lax.fori_loop(0, _B // 2, _edge2, 0)
    issue_scatter(p2)

    @pl.when(t + 3 < my)
    def _prefetch_idx():
      issue_idx(t + 3, p3)

  def _six(u, carry):
    for r in range(6):
      chunk_step(6 * u + r, r)
    return carry

  lax.fori_loop(0, my // 6, _six, 0)
  for r in range(my % 6):
    chunk_step(my - (my % 6) + r, r)
  wait_scatter(0)
  wait_scatter(1)

  plsc.subcore_barrier()
  pltpu.sync_copy(acc.at[pl.ds(s * rpt, rpt)], out.at[c, pl.ds(s * rpt, rpt)])

  @pl.when(s < ncr // 8)
  def _write_counts():
    pltpu.sync_copy(cacc.at[pl.ds(s * 8, 8)], outc.at[c, pl.ds(s * 8, 8)])


def kernel(H_t_in, ei_t, ew_t, W_rel1, b_rel1, W_root1, W_rel2, b_rel2,
           W_root2, W_rel3, b_rel3, W_root3, W_rel4, b_rel4, W_root4,
           W_rel5, b_rel5, W_root5, W_feat, b_feat, W_comb, b_comb,
           W_proj, b_proj):
  n = H_t_in.shape[0]
  e = ei_t.shape[1]
  nblk = _NPAD // _BLK
  n_chunks = e // _B
  ncr = _NPAD // 128

  w_all, bias_all = pl.pallas_call(
      _compose_body,
      out_shape=[
          jax.ShapeDtypeStruct((128, 1024), _F32),
          jax.ShapeDtypeStruct((2, 128), _F32),
      ],
  )(W_comb, W_rel1, W_rel2, W_rel3, W_rel4, W_rel5, W_root1, W_root2,
    W_root3, W_root4, W_root5, W_feat, W_proj, b_feat, b_comb, b_proj,
    b_rel1, b_rel2, b_rel3, b_rel4, b_rel5)

  x_pad = jnp.pad(H_t_in, ((0, _NPAD - n), (0, 0)))
  u6, p0, xf = pl.pallas_call(
      _transform_body,
      grid=(nblk,),
      in_specs=[
          pl.BlockSpec((_BLK, 128), lambda i: (i, 0)),
          pl.BlockSpec((128, 1024), lambda i: (0, 0)),
          pl.BlockSpec((2, 128), lambda i: (0, 0)),
      ],
      out_specs=[
          pl.BlockSpec((_BLK, 768), lambda i: (i, 0)),
          pl.BlockSpec((_BLK, 128), lambda i: (i, 0)),
          pl.BlockSpec((_BLK, 128), lambda i: (i, 0)),
      ],
      out_shape=[
          jax.ShapeDtypeStruct((_NPAD, 768), _F32),
          jax.ShapeDtypeStruct((_NPAD, 128), _F32),
          jax.ShapeDtypeStruct((_NPAD, 128), _F32),
      ],
  )(x_pad, w_all, bias_all)

  mesh = plsc.VectorSubcoreMesh(core_axis_name="c", subcore_axis_name="s")
  sc_agg = pl.kernel(
      functools.partial(_sc_body, n_chunks),
      out_type=[
          jax.ShapeDtypeStruct((_NC, _NPAD, 128), _F32),
          jax.ShapeDtypeStruct((_NC, ncr, 128), _F32),
      ],
      mesh=mesh,
      scratch_types=(
          [pltpu.VMEM_SHARED((_NPAD, 128), _F32),
           pltpu.VMEM_SHARED((ncr, 128), _F32)]
          + [pltpu.VMEM((18 * _B,), _F32)] * 3
          + [pltpu.VMEM((_B,), jnp.int32)] * 7
          + [pltpu.VMEM((_B, 768), _F32)] * 3
          + [pltpu.VMEM((_B, 128), _F32)] * 4
          + [pltpu.SemaphoreType.DMA] * 8
      ),
  )
  zer = jnp.zeros((_NPAD, 128), _F32)
  ew16 = jnp.concatenate(
      [ew_t.T, (ei_t[1] % 128).astype(_F32)[:, None],
       jnp.zeros((e, 9), _F32)], axis=1)
  edata = jnp.concatenate(
      [ei_t[0].astype(_F32).reshape(-1, _B),
       ei_t[1].astype(_F32).reshape(-1, _B),
       ew16.reshape(-1, 16 * _B)], axis=1).reshape(-1)
  parts, cnts = sc_agg(u6, edata, zer)

  out_pad = pl.pallas_call(
      _finish_body,
      grid=(nblk,),
      in_specs=[
          pl.BlockSpec((_NC, _BLK, 128), lambda i: (0, i, 0)),
          pl.BlockSpec((_NC, _BLK // 128, 128), lambda i: (0, i, 0)),
          pl.BlockSpec((_BLK, 128), lambda i: (i, 0)),
          pl.BlockSpec((_BLK, 128), lambda i: (i, 0)),
          pl.BlockSpec((128, 256), lambda i: (0, 0)),
      ],
      out_specs=pl.BlockSpec((_BLK, 128), lambda i: (i, 0)),
      out_shape=jax.ShapeDtypeStruct((_NPAD, 128), _F32),
  )(parts, cnts, p0, xf, W_proj)
  return out_pad[:n]


# speedup vs baseline: 1.0883x; 1.0883x over previous
import functools

import jax
import jax.numpy as jnp
from jax import lax
from jax.experimental import pallas as pl
from jax.experimental.pallas import tpu as pltpu
from jax.experimental.pallas import tpu_sc as plsc

_NC = 2
_NS = 16
_B = 16
_NPAD = 10240
_BLK = 1024

_F32 = jnp.float32


def _tdot(a, b):
  return lax.dot_general(a, b, (((0,), (1,)), ((), ())),
                         preferred_element_type=_F32)


def _rowdot(v, c):
  return lax.dot_general(v.reshape(1, -1), c, (((1,), (1,)), ((), ())),
                         preferred_element_type=_F32)


def _compose_body(wcomb, wr1, wr2, wr3, wr4, wr5, wo1, wo2, wo3, wo4, wo5,
                  wfeat, wproj, bfeat, bcomb, bproj, br1, br2, br3, br4, br5,
                  w_all_ref, bias_ref):
  wcomb = wcomb[...]
  cs = [wcomb[:, 128 * j:128 * (j + 1)] for j in range(7)]
  wrels = [wr1[...], wr2[...], wr3[...], wr4[...], wr5[...]]
  wroots = [wo1[...], wo2[...], wo3[...], wo4[...], wo5[...]]
  brels = [br1[...], br2[...], br3[...], br4[...], br5[...]]
  wrels.append(wrels[4])
  wroots.append(wroots[4])
  brels.append(brels[4])

  blocks = [_tdot(wrels[k], cs[k + 1]) for k in range(6)]
  gt = _tdot(wfeat[...], cs[0])
  for k in range(6):
    gt = gt + _tdot(wroots[k], cs[k + 1])
  wp1 = wproj[...][:, :128]
  ft = _tdot(wfeat[...], wp1)
  w_all_ref[...] = jnp.concatenate(blocks + [gt, ft], axis=1)

  cp = bcomb[...].reshape(1, -1) + _rowdot(bfeat[...], cs[0])
  for k in range(6):
    cp = cp + _rowdot(brels[k], cs[k + 1])
  dd = bproj[...].reshape(1, -1) + _rowdot(bfeat[...], wp1)
  bias_ref[...] = jnp.concatenate([cp, dd], axis=0)


def _transform_body(x_ref, w_ref, b_ref, u6_ref, p0_ref, xf_ref):
  out = jnp.dot(x_ref[...], w_ref[...], preferred_element_type=_F32)
  u6_ref[...] = out[:, :768]
  p0_ref[...] = out[:, 768:896] + b_ref[0][None, :]
  xf_ref[...] = out[:, 896:] + b_ref[1][None, :]


def _finish_body(parts_ref, cnts_ref, p0_ref, xf_ref, wp_ref, out_ref):
  s = parts_ref[0] + parts_ref[1]
  c4 = cnts_ref[0] + cnts_ref[1]
  nr = _BLK // 128
  sel = (lax.broadcasted_iota(jnp.int32, (_BLK, nr), 0) // 128 ==
         lax.broadcasted_iota(jnp.int32, (_BLK, nr), 1)).astype(_F32)
  rep = jnp.dot(sel, c4, preferred_element_type=_F32)
  pmod = lax.broadcasted_iota(jnp.int32, (_BLK, 128), 0) % 128
  lane = lax.broadcasted_iota(jnp.int32, (_BLK, 128), 1)
  cnt_col = jnp.sum(jnp.where(pmod == lane, rep, 0.0), axis=1, keepdims=True)
  cnt = jnp.maximum(cnt_col, 1.0)
  proj = jnp.maximum(p0_ref[...] + s / cnt, 0.0)
  o = xf_ref[...] + lax.dot_general(
      proj, wp_ref[...][:, 128:], (((1,), (1,)), ((), ())),
      preferred_element_type=_F32)
  nrm = jnp.sqrt(jnp.sum(o * o, axis=1, keepdims=True))
  out_ref[...] = o / jnp.maximum(nrm, 1e-12)


def _sc_body(n_chunks, u6, edata, zer, out, outc,
             acc, cacc, fbuf0, fbuf1, srcv0, srcv1,
             dstv0, dstv1, ddv0, ddv1, rows0, rows1, msg0, msg1,
             msg20, msg21, esem0, esem1, gsem0, gsem1, ssem0, ssem1):
  c = lax.axis_index("c")
  s = lax.axis_index("s")
  w = c * _NS + s
  nw = _NC * _NS
  rpt = _NPAD // _NS
  ncr = _NPAD // 128
  my = n_chunks // nw

  fbufs = [fbuf0, fbuf1]
  srcvs = [srcv0, srcv1]
  dstvs = [dstv0, dstv1]
  ddvs = [ddv0, ddv1]
  rowss = [rows0, rows1]
  msgs = [msg0, msg1]
  msg2s = [msg20, msg21]
  esems = [esem0, esem1]
  gsems = [gsem0, gsem1]
  ssems = [ssem0, ssem1]

  pltpu.sync_copy(zer.at[pl.ds(s * rpt, rpt)], acc.at[pl.ds(s * rpt, rpt)])

  @pl.when(s < ncr // 8)
  def _zero_counts():
    pltpu.sync_copy(zer.at[pl.ds(s * 8, 8)], cacc.at[pl.ds(s * 8, 8)])

  plsc.subcore_barrier()

  iota16 = lax.convert_element_type(
      lax.broadcasted_iota(jnp.int32, (16,), 0), _F32)

  rec = _B * 18

  def issue_idx(t, b):
    lo = (w + t * nw) * rec
    pltpu.async_copy(edata.at[pl.ds(lo, rec)], fbufs[b], esems[b])

  def wait_idx(b):
    pltpu.make_async_copy(edata.at[pl.ds(0, rec)], fbufs[b],
                          esems[b]).wait()

  def issue_gather(b):
    srcvs[b][pl.ds(0, 16)] = lax.convert_element_type(
        fbufs[b][pl.ds(0, 16)], jnp.int32)
    pltpu.async_copy(u6.at[srcvs[b]], rowss[b], gsems[b])

  def wait_gather(b):
    pltpu.make_async_copy(u6.at[srcvs[b]], rowss[b], gsems[b]).wait()

  def issue_scatter(b):
    pltpu.async_copy(msgs[b], acc.at[dstvs[b]], ssems[b], add=True)
    pltpu.async_copy(msg2s[b], cacc.at[ddvs[b]], ssems[b], add=True)

  def wait_scatter(b):
    pltpu.make_async_copy(msgs[b], acc.at[dstvs[b]], ssems[b]).wait()
    pltpu.make_async_copy(msg2s[b], cacc.at[ddvs[b]], ssems[b]).wait()

  issue_idx(0, 0)
  issue_idx(1, 1)
  wait_idx(0)
  issue_gather(0)

  def chunk_step(t, p):
    q = 1 - p

    @pl.when(t + 1 < my)
    def _prefetch_gather():
      wait_idx(q)
      issue_gather(q)

    wait_gather(p)

    @pl.when(t >= 2)
    def _drain_scatter():
      wait_scatter(p)

    dpart = lax.convert_element_type(fbufs[p][pl.ds(16, 16)], jnp.int32)
    dstvs[p][pl.ds(0, 16)] = dpart
    ddvs[p][pl.ds(0, 16)] = lax.shift_right_logical(dpart, 7)

    def _edge(i, inner):
      wrow = fbufs[p][pl.ds(32 + i * 16, 16)]
      dm = wrow[6]
      for j in range(8):
        v = wrow[0] * rowss[p][i, pl.ds(j * 16, 16)]
        for k in range(1, 6):
          v = v + wrow[k] * rowss[p][i, pl.ds(k * 128 + j * 16, 16)]
        msgs[p][i, pl.ds(j * 16, 16)] = v
        msg2s[p][i, pl.ds(j * 16, 16)] = jnp.where(
            iota16 + (16.0 * j) == dm, 1.0, 0.0)
      return inner

    lax.fori_loop(0, _B, _edge, 0)
    issue_scatter(p)

    @pl.when(t + 2 < my)
    def _prefetch_idx():
      issue_idx(t + 2, p)

  def _pair(u, carry):
    chunk_step(2 * u, 0)
    chunk_step(2 * u + 1, 1)
    return carry

  lax.fori_loop(0, my // 2, _pair, 0)
  if my % 2:
    chunk_step(my - 1, 0)
  wait_scatter(1 - (my % 2))
  wait_scatter(my % 2)

  plsc.subcore_barrier()
  pltpu.sync_copy(acc.at[pl.ds(s * rpt, rpt)], out.at[c, pl.ds(s * rpt, rpt)])

  @pl.when(s < ncr // 8)
  def _write_counts():
    pltpu.sync_copy(cacc.at[pl.ds(s * 8, 8)], outc.at[c, pl.ds(s * 8, 8)])


def kernel(H_t_in, ei_t, ew_t, W_rel1, b_rel1, W_root1, W_rel2, b_rel2,
           W_root2, W_rel3, b_rel3, W_root3, W_rel4, b_rel4, W_root4,
           W_rel5, b_rel5, W_root5, W_feat, b_feat, W_comb, b_comb,
           W_proj, b_proj):
  n = H_t_in.shape[0]
  e = ei_t.shape[1]
  nblk = _NPAD // _BLK
  n_chunks = e // _B
  ncr = _NPAD // 128

  w_all, bias_all = pl.pallas_call(
      _compose_body,
      out_shape=[
          jax.ShapeDtypeStruct((128, 1024), _F32),
          jax.ShapeDtypeStruct((2, 128), _F32),
      ],
  )(W_comb, W_rel1, W_rel2, W_rel3, W_rel4, W_rel5, W_root1, W_root2,
    W_root3, W_root4, W_root5, W_feat, W_proj, b_feat, b_comb, b_proj,
    b_rel1, b_rel2, b_rel3, b_rel4, b_rel5)

  x_pad = jnp.pad(H_t_in, ((0, _NPAD - n), (0, 0)))
  u6, p0, xf = pl.pallas_call(
      _transform_body,
      grid=(nblk,),
      in_specs=[
          pl.BlockSpec((_BLK, 128), lambda i: (i, 0)),
          pl.BlockSpec((128, 1024), lambda i: (0, 0)),
          pl.BlockSpec((2, 128), lambda i: (0, 0)),
      ],
      out_specs=[
          pl.BlockSpec((_BLK, 768), lambda i: (i, 0)),
          pl.BlockSpec((_BLK, 128), lambda i: (i, 0)),
          pl.BlockSpec((_BLK, 128), lambda i: (i, 0)),
      ],
      out_shape=[
          jax.ShapeDtypeStruct((_NPAD, 768), _F32),
          jax.ShapeDtypeStruct((_NPAD, 128), _F32),
          jax.ShapeDtypeStruct((_NPAD, 128), _F32),
      ],
  )(x_pad, w_all, bias_all)

  mesh = plsc.VectorSubcoreMesh(core_axis_name="c", subcore_axis_name="s")
  sc_agg = pl.kernel(
      functools.partial(_sc_body, n_chunks),
      out_type=[
          jax.ShapeDtypeStruct((_NC, _NPAD, 128), _F32),
          jax.ShapeDtypeStruct((_NC, ncr, 128), _F32),
      ],
      mesh=mesh,
      scratch_types=(
          [pltpu.VMEM_SHARED((_NPAD, 128), _F32),
           pltpu.VMEM_SHARED((ncr, 128), _F32)]
          + [pltpu.VMEM((18 * _B,), _F32)] * 2
          + [pltpu.VMEM((_B,), jnp.int32)] * 6
          + [pltpu.VMEM((_B, 768), _F32)] * 2
          + [pltpu.VMEM((_B, 128), _F32)] * 4
          + [pltpu.SemaphoreType.DMA] * 6
      ),
  )
  zer = jnp.zeros((_NPAD, 128), _F32)
  ew16 = jnp.concatenate(
      [ew_t.T, (ei_t[1] % 128).astype(_F32)[:, None],
       jnp.zeros((e, 9), _F32)], axis=1)
  edata = jnp.concatenate(
      [ei_t[0].astype(_F32).reshape(-1, _B),
       ei_t[1].astype(_F32).reshape(-1, _B),
       ew16.reshape(-1, 16 * _B)], axis=1).reshape(-1)
  parts, cnts = sc_agg(u6, edata, zer)

  out_pad = pl.pallas_call(
      _finish_body,
      grid=(nblk,),
      in_specs=[
          pl.BlockSpec((_NC, _BLK, 128), lambda i: (0, i, 0)),
          pl.BlockSpec((_NC, _BLK // 128, 128), lambda i: (0, i, 0)),
          pl.BlockSpec((_BLK, 128), lambda i: (i, 0)),
          pl.BlockSpec((_BLK, 128), lambda i: (i, 0)),
          pl.BlockSpec((128, 256), lambda i: (0, 0)),
      ],
      out_specs=pl.BlockSpec((_BLK, 128), lambda i: (i, 0)),
      out_shape=jax.ShapeDtypeStruct((_NPAD, 128), _F32),
  )(parts, cnts, p0, xf, W_proj)
  return out_pad[:n]


# speedup vs baseline: 1.1639x; 1.0695x over previous
import functools

import jax
import jax.numpy as jnp
from jax import lax
from jax.experimental import pallas as pl
from jax.experimental.pallas import tpu as pltpu
from jax.experimental.pallas import tpu_sc as plsc

_NC = 2
_NS = 16
_B = 16
_NPAD = 10240
_NACC = 10368
_BLK = 1024

_F32 = jnp.float32


def _tdot(a, b):
  return lax.dot_general(a, b, (((0,), (1,)), ((), ())),
                         preferred_element_type=_F32)


def _rowdot(v, c):
  return lax.dot_general(v.reshape(1, -1), c, (((1,), (1,)), ((), ())),
                         preferred_element_type=_F32)


def _compose_body(wcomb, wr1, wr2, wr3, wr4, wr5, wo1, wo2, wo3, wo4, wo5,
                  wfeat, wproj, bfeat, bcomb, bproj, br1, br2, br3, br4, br5,
                  w_all_ref, bias_ref):
  wcomb = wcomb[...]
  cs = [wcomb[:, 128 * j:128 * (j + 1)] for j in range(7)]
  wrels = [wr1[...], wr2[...], wr3[...], wr4[...], wr5[...]]
  wroots = [wo1[...], wo2[...], wo3[...], wo4[...], wo5[...]]
  brels = [br1[...], br2[...], br3[...], br4[...], br5[...]]
  wrels.append(wrels[4])
  wroots.append(wroots[4])
  brels.append(brels[4])

  blocks = [_tdot(wrels[k], cs[k + 1]) for k in range(6)]
  gt = _tdot(wfeat[...], cs[0])
  for k in range(6):
    gt = gt + _tdot(wroots[k], cs[k + 1])
  wp1 = wproj[...][:, :128]
  ft = _tdot(wfeat[...], wp1)
  w_all_ref[...] = jnp.concatenate(blocks + [gt, ft], axis=1)

  cp = bcomb[...].reshape(1, -1) + _rowdot(bfeat[...], cs[0])
  for k in range(6):
    cp = cp + _rowdot(brels[k], cs[k + 1])
  dd = bproj[...].reshape(1, -1) + _rowdot(bfeat[...], wp1)
  bias_ref[...] = jnp.concatenate([cp, dd], axis=0)


def _transform_body(x_ref, w_ref, b_ref, u6_ref, p0_ref, xf_ref):
  out = jnp.dot(x_ref[...], w_ref[...], preferred_element_type=_F32)
  u6_ref[...] = out[:, :768]
  p0_ref[...] = out[:, 768:896] + b_ref[0][None, :]
  xf_ref[...] = out[:, 896:] + b_ref[1][None, :]


def _finish_body(parts_ref, cnts_ref, p0_ref, xf_ref, wp_ref, out_ref):
  s = parts_ref[0] + parts_ref[1]
  c4 = cnts_ref[0] + cnts_ref[1]
  nr = _BLK // 128
  sel = (lax.broadcasted_iota(jnp.int32, (_BLK, nr), 0) // 128 ==
         lax.broadcasted_iota(jnp.int32, (_BLK, nr), 1)).astype(_F32)
  rep = jnp.dot(sel, c4, preferred_element_type=_F32)
  pmod = lax.broadcasted_iota(jnp.int32, (_BLK, 128), 0) % 128
  lane = lax.broadcasted_iota(jnp.int32, (_BLK, 128), 1)
  cnt_col = jnp.sum(jnp.where(pmod == lane, rep, 0.0), axis=1, keepdims=True)
  cnt = jnp.maximum(cnt_col, 1.0)
  proj = jnp.maximum(p0_ref[...] + s / cnt, 0.0)
  o = xf_ref[...] + lax.dot_general(
      proj, wp_ref[...][:, 128:], (((1,), (1,)), ((), ())),
      preferred_element_type=_F32)
  nrm = jnp.sqrt(jnp.sum(o * o, axis=1, keepdims=True))
  out_ref[...] = o / jnp.maximum(nrm, 1e-12)


def _sc_body(n_chunks, u6, eint, eflt, zer, out,
             acc, ibuf0, ibuf1, fbuf0, fbuf1, srcv0, srcv1,
             dstv0, dstv1, rows0, rows1, msg0, msg1,
             esem0, esem1, gsem0, gsem1, ssem0, ssem1):
  c = lax.axis_index("c")
  s = lax.axis_index("s")
  w = c * _NS + s
  nw = _NC * _NS
  rpt = _NACC // _NS
  my = n_chunks // nw

  ibufs = [ibuf0, ibuf1]
  fbufs = [fbuf0, fbuf1]
  srcvs = [srcv0, srcv1]
  dstvs = [dstv0, dstv1]
  rowss = [rows0, rows1]
  msgs = [msg0, msg1]
  esems = [esem0, esem1]
  gsems = [gsem0, gsem1]
  ssems = [ssem0, ssem1]

  pltpu.sync_copy(zer.at[pl.ds(s * rpt, rpt)], acc.at[pl.ds(s * rpt, rpt)])

  plsc.subcore_barrier()

  iota16 = lax.convert_element_type(
      lax.broadcasted_iota(jnp.int32, (16,), 0), _F32)

  def issue_idx(t, b):
    q = w + t * nw
    pltpu.async_copy(eint.at[pl.ds(q * (2 * _B), 2 * _B)], ibufs[b],
                     esems[b])
    pltpu.async_copy(eflt.at[pl.ds(q * (16 * _B), 16 * _B)], fbufs[b],
                     esems[b])

  def wait_idx(b):
    pltpu.make_async_copy(eint.at[pl.ds(0, 2 * _B)], ibufs[b],
                          esems[b]).wait()
    pltpu.make_async_copy(eflt.at[pl.ds(0, 16 * _B)], fbufs[b],
                          esems[b]).wait()

  def issue_gather(b):
    srcvs[b][pl.ds(0, 16)] = ibufs[b][pl.ds(0, 16)]
    pltpu.async_copy(u6.at[srcvs[b]], rowss[b], gsems[b])

  def wait_gather(b):
    pltpu.make_async_copy(u6.at[srcvs[b]], rowss[b], gsems[b]).wait()

  def issue_scatter(b):
    pltpu.async_copy(msgs[b], acc.at[dstvs[b]], ssems[b], add=True)

  def wait_scatter(b):
    pltpu.make_async_copy(msgs[b], acc.at[dstvs[b]], ssems[b]).wait()

  issue_idx(0, 0)
  issue_idx(1, 1)
  wait_idx(0)
  issue_gather(0)

  def chunk_step(t, p):
    q = 1 - p

    @pl.when(t + 1 < my)
    def _prefetch_gather():
      wait_idx(q)
      issue_gather(q)

    wait_gather(p)

    @pl.when(t >= 2)
    def _drain_scatter():
      wait_scatter(p)

    dpart = ibufs[p][pl.ds(16, 16)]
    dstvs[p][pl.ds(0, 16)] = dpart
    dstvs[p][pl.ds(16, 16)] = (
        lax.shift_right_logical(dpart, 7) + _NPAD)

    def _edge(i, inner):
      wrow = fbufs[p][pl.ds(i * 16, 16)]
      dm = wrow[6]
      for j in range(8):
        v = wrow[0] * rowss[p][i, pl.ds(j * 16, 16)]
        for k in range(1, 6):
          v = v + wrow[k] * rowss[p][i, pl.ds(k * 128 + j * 16, 16)]
        msgs[p][i, pl.ds(j * 16, 16)] = v
        msgs[p][16 + i, pl.ds(j * 16, 16)] = jnp.where(
            iota16 + (16.0 * j) == dm, 1.0, 0.0)
      return inner

    lax.fori_loop(0, _B, _edge, 0)
    issue_scatter(p)

    @pl.when(t + 2 < my)
    def _prefetch_idx():
      issue_idx(t + 2, p)

  def _pair(u, carry):
    chunk_step(2 * u, 0)
    chunk_step(2 * u + 1, 1)
    return carry

  lax.fori_loop(0, my // 2, _pair, 0)
  if my % 2:
    chunk_step(my - 1, 0)
  wait_scatter(1 - (my % 2))
  wait_scatter(my % 2)

  plsc.subcore_barrier()
  pltpu.sync_copy(acc.at[pl.ds(s * rpt, rpt)], out.at[c, pl.ds(s * rpt, rpt)])


def kernel(H_t_in, ei_t, ew_t, W_rel1, b_rel1, W_root1, W_rel2, b_rel2,
           W_root2, W_rel3, b_rel3, W_root3, W_rel4, b_rel4, W_root4,
           W_rel5, b_rel5, W_root5, W_feat, b_feat, W_comb, b_comb,
           W_proj, b_proj):
  n = H_t_in.shape[0]
  e = ei_t.shape[1]
  nblk = _NPAD // _BLK
  n_chunks = e // _B
  ncr = _NPAD // 128

  w_all, bias_all = pl.pallas_call(
      _compose_body,
      out_shape=[
          jax.ShapeDtypeStruct((128, 1024), _F32),
          jax.ShapeDtypeStruct((2, 128), _F32),
      ],
  )(W_comb, W_rel1, W_rel2, W_rel3, W_rel4, W_rel5, W_root1, W_root2,
    W_root3, W_root4, W_root5, W_feat, W_proj, b_feat, b_comb, b_proj,
    b_rel1, b_rel2, b_rel3, b_rel4, b_rel5)

  x_pad = jnp.pad(H_t_in, ((0, _NPAD - n), (0, 0)))
  u6, p0, xf = pl.pallas_call(
      _transform_body,
      grid=(nblk,),
      in_specs=[
          pl.BlockSpec((_BLK, 128), lambda i: (i, 0)),
          pl.BlockSpec((128, 1024), lambda i: (0, 0)),
          pl.BlockSpec((2, 128), lambda i: (0, 0)),
      ],
      out_specs=[
          pl.BlockSpec((_BLK, 768), lambda i: (i, 0)),
          pl.BlockSpec((_BLK, 128), lambda i: (i, 0)),
          pl.BlockSpec((_BLK, 128), lambda i: (i, 0)),
      ],
      out_shape=[
          jax.ShapeDtypeStruct((_NPAD, 768), _F32),
          jax.ShapeDtypeStruct((_NPAD, 128), _F32),
          jax.ShapeDtypeStruct((_NPAD, 128), _F32),
      ],
  )(x_pad, w_all, bias_all)

  mesh = plsc.VectorSubcoreMesh(core_axis_name="c", subcore_axis_name="s")
  sc_agg = pl.kernel(
      functools.partial(_sc_body, n_chunks),
      out_type=jax.ShapeDtypeStruct((_NC, _NACC, 128), _F32),
      mesh=mesh,
      scratch_types=(
          [pltpu.VMEM_SHARED((_NACC, 128), _F32)]
          + [pltpu.VMEM((2 * _B,), jnp.int32)] * 2
          + [pltpu.VMEM((16 * _B,), _F32)] * 2
          + [pltpu.VMEM((_B,), jnp.int32)] * 2
          + [pltpu.VMEM((2 * _B,), jnp.int32)] * 2
          + [pltpu.VMEM((_B, 768), _F32)] * 2
          + [pltpu.VMEM((2 * _B, 128), _F32)] * 2
          + [pltpu.SemaphoreType.DMA] * 6
      ),
  )
  zer = jnp.zeros((_NACC, 128), _F32)
  eflt = jnp.concatenate(
      [ew_t.T, (ei_t[1] % 128).astype(_F32)[:, None],
       jnp.zeros((e, 9), _F32)], axis=1).reshape(-1)
  eint = jnp.concatenate(
      [ei_t[0].reshape(-1, _B), ei_t[1].reshape(-1, _B)],
      axis=1).reshape(-1)
  parts = sc_agg(u6, eint, eflt, zer)

  out_pad = pl.pallas_call(
      _finish_body,
      grid=(nblk,),
      in_specs=[
          pl.BlockSpec((_NC, _BLK, 128), lambda i: (0, i, 0)),
          pl.BlockSpec((_NC, 8, 128), lambda i: (0, _NPAD // 8 + i, 0)),
          pl.BlockSpec((_BLK, 128), lambda i: (i, 0)),
          pl.BlockSpec((_BLK, 128), lambda i: (i, 0)),
          pl.BlockSpec((128, 256), lambda i: (0, 0)),
      ],
      out_specs=pl.BlockSpec((_BLK, 128), lambda i: (i, 0)),
      out_shape=jax.ShapeDtypeStruct((_NPAD, 128), _F32),
  )(parts, parts, p0, xf, W_proj)
  return out_pad[:n]
